# Initial kernel scaffold; baseline (speedup 1.0000x reference)
import jax
import jax.numpy as jnp
from jax.experimental import pallas as pl


def kernel(vfeat, in_src, in_dst, con_src, con_dst, W_in, b_in, W_v, W_e, W_0):
    raise NotImplementedError("write your pallas kernel here")


# speedup vs baseline: 1.5666x; 1.5666x over previous
import functools

import jax
import jax.numpy as jnp
from jax import lax
from jax.experimental import pallas as pl
from jax.experimental.pallas import tpu as pltpu
from jax.experimental.pallas import tpu_sc as plsc

N = 10000
H = 5000
NP = 10240
HP = 5120
E = 160000
D = 128

NC = 2
NS = 16
NW = NC * NS
C = 128
EP = 163840
EPW = EP // NW
NCH = EPW // C

NSTR = NP // NS
HSTR = HP // NS
ZC = 32

F32 = jnp.float32
I32 = jnp.int32
_mesh = plsc.VectorSubcoreMesh(core_axis_name="c", subcore_axis_name="s")


def _make_gather_scatter(T, STR):
    nz = STR // ZC

    @functools.partial(
        pl.kernel,
        out_type=[
            jax.ShapeDtypeStruct((EP, D), F32),
            jax.ShapeDtypeStruct((NC, T, D), F32),
        ],
        mesh=_mesh,
        scratch_types=[
            pltpu.VMEM((C,), I32),
            pltpu.VMEM((8, C), I32),
            pltpu.VMEM((C, D), F32),
            pltpu.VMEM((ZC, D), F32),
            pltpu.VMEM_SHARED((T, D), F32),
            pltpu.SemaphoreType.DMA,
        ],
    )
    def k(src_h, dst_h, tab_h, zn_h, g_o, acc_o, ii, jj2, rows, zb, acc_sp,
          sem):
        c = lax.axis_index("c")
        s = lax.axis_index("s")
        base = (c * NS + s) * EPW

        pltpu.sync_copy(zn_h, zb)
        for t in range(nz):
            pltpu.sync_copy(zb, acc_sp.at[pl.ds(s * STR + t * ZC, ZC)])
        plsc.subcore_barrier()

        def chunk(q, carry):
            gb = base + q * C
            jj = jj2.at[0]
            pltpu.sync_copy(src_h.at[pl.ds(gb, C)], ii)
            pltpu.sync_copy(dst_h.at[pl.ds(gb, C)], jj)
            pltpu.async_copy(tab_h.at[ii], rows, sem).wait()
            pltpu.sync_copy(rows, acc_sp.at[jj], add=True)
            pltpu.sync_copy(rows, g_o.at[pl.ds(gb, C)])
            return carry

        lax.fori_loop(0, NCH, chunk, 0)

        plsc.subcore_barrier()
        for t in range(nz):
            pltpu.sync_copy(acc_sp.at[pl.ds(s * STR + t * ZC, ZC)], zb)
            pltpu.sync_copy(zb, acc_o.at[c, pl.ds(s * STR + t * ZC, ZC)])

    return k


_sc_gather_scatter_h = _make_gather_scatter(HP, HSTR)


def _make_count(T, STR):
    nz = STR // ZC

    @functools.partial(
        pl.kernel,
        out_type=jax.ShapeDtypeStruct((NC, T, D), F32),
        mesh=_mesh,
        scratch_types=[
            pltpu.VMEM((8, C), I32),
            pltpu.VMEM((C, D), F32),
            pltpu.VMEM((ZC, D), F32),
            pltpu.VMEM_SHARED((T, D), F32),
        ],
    )
    def k(idx_h, ones_h, zn_h, acc_o, jj2, ones_v, zb, acc_sp):
        c = lax.axis_index("c")
        s = lax.axis_index("s")
        base = (c * NS + s) * EPW

        pltpu.sync_copy(zn_h, zb)
        pltpu.sync_copy(ones_h, ones_v)
        for t in range(nz):
            pltpu.sync_copy(zb, acc_sp.at[pl.ds(s * STR + t * ZC, ZC)])
        plsc.subcore_barrier()

        def chunk(q, carry):
            gb = base + q * C
            jj = jj2.at[0]
            pltpu.sync_copy(idx_h.at[pl.ds(gb, C)], jj)
            pltpu.sync_copy(ones_v, acc_sp.at[jj], add=True)
            return carry

        lax.fori_loop(0, NCH, chunk, 0)

        plsc.subcore_barrier()
        for t in range(nz):
            pltpu.sync_copy(acc_sp.at[pl.ds(s * STR + t * ZC, ZC)], zb)
            pltpu.sync_copy(zb, acc_o.at[c, pl.ds(s * STR + t * ZC, ZC)])

    return k


_sc_count_n = _make_count(NP, NSTR)
_sc_count_h = _make_count(HP, HSTR)


@functools.partial(
    pl.kernel,
    out_type=[
        jax.ShapeDtypeStruct((EP, D), F32),
        jax.ShapeDtypeStruct((EP, D), F32),
    ],
    mesh=_mesh,
    scratch_types=[
        pltpu.VMEM((C,), I32),
        pltpu.VMEM((C,), I32),
        pltpu.VMEM((C, D), F32),
        pltpu.VMEM((C, D), F32),
        pltpu.SemaphoreType.DMA,
    ],
)
def _sc_dual_gather(srca_h, srcb_h, taba_h, tabb_h, ga_o, gb_o,
                    ii, jj, ra, rb, sem):
    c = lax.axis_index("c")
    s = lax.axis_index("s")
    base = (c * NS + s) * EPW

    def chunk(q, carry):
        gb = base + q * C
        pltpu.sync_copy(srca_h.at[pl.ds(gb, C)], ii)
        pltpu.sync_copy(srcb_h.at[pl.ds(gb, C)], jj)
        d1 = pltpu.async_copy(taba_h.at[ii], ra, sem)
        d2 = pltpu.async_copy(tabb_h.at[jj], rb, sem)
        d1.wait()
        d2.wait()
        pltpu.sync_copy(ra, ga_o.at[pl.ds(gb, C)])
        pltpu.sync_copy(rb, gb_o.at[pl.ds(gb, C)])
        return carry

    lax.fori_loop(0, NCH, chunk, 0)


def _make_scatter(T, STR):
    nz = STR // ZC

    @functools.partial(
        pl.kernel,
        out_type=jax.ShapeDtypeStruct((NC, T, D), F32),
        mesh=_mesh,
        scratch_types=[
            pltpu.VMEM((8, C), I32),
            pltpu.VMEM((C, D), F32),
            pltpu.VMEM((ZC, D), F32),
            pltpu.VMEM_SHARED((T, D), F32),
        ],
    )
    def k(idx_h, u_h, zn_h, acc_o, jj2, uv, zb, acc_sp):
        c = lax.axis_index("c")
        s = lax.axis_index("s")
        base = (c * NS + s) * EPW

        pltpu.sync_copy(zn_h, zb)
        for t in range(nz):
            pltpu.sync_copy(zb, acc_sp.at[pl.ds(s * STR + t * ZC, ZC)])
        plsc.subcore_barrier()

        def chunk(q, carry):
            gb = base + q * C
            jj = jj2.at[0]
            pltpu.sync_copy(idx_h.at[pl.ds(gb, C)], jj)
            pltpu.sync_copy(u_h.at[pl.ds(gb, C)], uv)
            pltpu.sync_copy(uv, acc_sp.at[jj], add=True)
            return carry

        lax.fori_loop(0, NCH, chunk, 0)

        plsc.subcore_barrier()
        for t in range(nz):
            pltpu.sync_copy(acc_sp.at[pl.ds(s * STR + t * ZC, ZC)], zb)
            pltpu.sync_copy(zb, acc_o.at[c, pl.ds(s * STR + t * ZC, ZC)])

    return k


_sc_scatter_n = _make_scatter(NP, NSTR)
_sc_scatter_h = _make_scatter(HP, HSTR)


BR_N = 400
BR_H = 320
BR_E = 2048


def _tc1_body(v_ref, wi_ref, b_ref, wv_ref, w0_ref, r_ref, a1_ref, q0_ref):
    t = jnp.dot(v_ref[...], wi_ref[...], preferred_element_type=F32) + b_ref[...]
    r = jnp.maximum(t, 0.0)
    q0 = jnp.dot(r, w0_ref[...], preferred_element_type=F32)
    r_ref[...] = r
    q0_ref[...] = q0
    a1_ref[...] = jnp.dot(r, wv_ref[...], preferred_element_type=F32) + q0


def _tc2_body(sp_ref, cp_ref, we_ref, b1_ref):
    ssum = sp_ref[0] + sp_ref[1]
    cnt = (cp_ref[0] + cp_ref[1])[:, 0:1]
    m = ssum / jnp.maximum(cnt, 1.0)
    b1_ref[...] = jnp.dot(m, we_ref[...], preferred_element_type=F32)


def _tc3a_body(q0_ref, svp_ref, cvp_ref, wv_ref, a2_ref):
    ssum = svp_ref[0] + svp_ref[1]
    cnt = (cvp_ref[0] + cvp_ref[1])[:, 0:1]
    m = ssum / jnp.maximum(cnt, 1.0)
    a2_ref[...] = jnp.dot(m, wv_ref[...], preferred_element_type=F32) + q0_ref[...]


def _tcu_body(g_ref, ga_ref, gb_ref, u_ref):
    u_ref[...] = g_ref[...] + jnp.maximum(ga_ref[...] + gb_ref[...], 0.0)


def _full(shape):
    return pl.BlockSpec(shape, lambda i: (0,) * len(shape))


def _rows(br, w):
    return pl.BlockSpec((br, w), lambda i: (i, 0))


def _prows(br, w):
    return pl.BlockSpec((NC, br, w), lambda i: (0, i, 0))


_tc1 = pl.pallas_call(
    _tc1_body,
    grid=(N // BR_N,),
    in_specs=[_rows(BR_N, D), _full((D, D)), _full((1, D)), _full((D, D)),
              _full((D, D))],
    out_specs=[_rows(BR_N, D), _rows(BR_N, D), _rows(BR_N, D)],
    out_shape=[jax.ShapeDtypeStruct((N, D), F32),
               jax.ShapeDtypeStruct((N, D), F32),
               jax.ShapeDtypeStruct((N, D), F32)],
)

_tc2 = pl.pallas_call(
    _tc2_body,
    grid=(HP // BR_H,),
    in_specs=[_prows(BR_H, D), _prows(BR_H, D), _full((D, D))],
    out_specs=_rows(BR_H, D),
    out_shape=jax.ShapeDtypeStruct((HP, D), F32),
)

_tc3a = pl.pallas_call(
    _tc3a_body,
    grid=(N // BR_N,),
    in_specs=[_rows(BR_N, D), _prows(BR_N, D), _prows(BR_N, D),
              _full((D, D))],
    out_specs=_rows(BR_N, D),
    out_shape=jax.ShapeDtypeStruct((N, D), F32),
)

_tcu = pl.pallas_call(
    _tcu_body,
    grid=(EP // BR_E,),
    in_specs=[_rows(BR_E, D), _rows(BR_E, D), _rows(BR_E, D)],
    out_specs=_rows(BR_E, D),
    out_shape=jax.ShapeDtypeStruct((EP, D), F32),
)


def kernel(vfeat, in_src, in_dst, con_src, con_dst, W_in, b_in, W_v, W_e, W_0):
    del con_src, con_dst
    i = in_src.astype(I32)
    j = in_dst.astype(I32)
    ip = jnp.concatenate([i, jnp.full((EP - E,), N, I32)])
    jp = jnp.concatenate([j, jnp.full((EP - E,), H, I32)])
    b2 = b_in.reshape(1, D).astype(F32)

    R, A1, Q0 = _tc1(vfeat.astype(F32), W_in, b2, W_v, W_0)
    Rp = jnp.pad(R, ((0, NP - N), (0, 0)))
    A1p = jnp.pad(A1, ((0, NP - N), (0, 0)))

    zn = jnp.zeros((ZC, D), F32)
    ones_c = jnp.ones((C, D), F32)

    G1, se1p = _sc_gather_scatter_h(ip, jp, Rp, zn)
    cvp = _sc_count_n(ip, ones_c, zn)
    cep = _sc_count_h(jp, ones_c, zn)

    B1 = _tc2(se1p, cep, W_e)

    GA1, GB1 = _sc_dual_gather(ip, jp, A1p, B1)
    u1p = _tcu(G1, GA1, GB1)

    svp = _sc_scatter_n(ip, u1p, zn)
    sep = _sc_scatter_h(jp, u1p, zn)

    A2 = _tc3a(Q0, svp, cvp, W_v)
    B2 = _tc2(sep, cep, W_e)
    A2p = jnp.pad(A2, ((0, NP - N), (0, 0)))

    GA2, GB2 = _sc_dual_gather(ip, jp, A2p, B2)
    out = _tcu(u1p, GA2, GB2)[:E]
    return out, jnp.arange(E, dtype=I32)
